# baseline (speedup 1.0000x reference)
import functools
import math

import jax
import jax.numpy as jnp
import numpy as np
from jax import lax
from jax.experimental import pallas as pl
from jax.experimental.pallas import tpu as pltpu
from jax.experimental.pallas import tpu_sc as plsc

_D_MODEL = 1024
_MAX_LEN = 2048
_BATCH = 4
_SEQ = 2048


def _pe_table(max_len, d_model):
    pe = np.zeros((max_len, d_model), dtype=np.float32)
    pos = np.arange(max_len, dtype=np.float32)[:, None]
    div = np.exp(
        np.arange(0, d_model, 2, dtype=np.float32) * (-math.log(10000.0) / d_model)
    )
    pe[:, 0::2] = np.sin(pos * div)
    pe[:, 1::2] = np.cos(pos * div[: d_model // 2])
    return pe


_PE = _pe_table(_MAX_LEN, _D_MODEL)

_NC = 2
_NS = 16
_NW = _NC * _NS
_LANES = 16
_S_SC = 512
_ROWS_W = _S_SC // _NW
_ND = _D_MODEL // _LANES
_DC = 16

_mesh = plsc.VectorSubcoreMesh(core_axis_name="c", subcore_axis_name="s")


@functools.partial(
    pl.kernel,
    mesh=_mesh,
    out_type=jax.ShapeDtypeStruct((_BATCH, _SEQ, _D_MODEL), jnp.float32),
    scratch_types=[
        pltpu.VMEM((2, _D_MODEL), jnp.float32),
        pltpu.VMEM((_BATCH, _ROWS_W + _LANES), jnp.int32),
        pltpu.VMEM((_ROWS_W, _D_MODEL), jnp.float32),
        pltpu.VMEM((2, _ROWS_W, _D_MODEL), jnp.float32),
        pltpu.SemaphoreType.DMA,
        pltpu.SemaphoreType.DMA,
    ],
)
def _sc_embed(x_hbm, tab_hbm, pe_hbm, out_hbm, tb_v, x_v, pe_v, o_v, o_sem, i_sem):
    wid = lax.axis_index("s") * _NC + lax.axis_index("c")
    l0 = wid * _ROWS_W
    c_tab = pltpu.async_copy(tab_hbm, tb_v, i_sem)
    c_x = pltpu.async_copy(
        x_hbm.at[:, pl.ds(l0, _ROWS_W)], x_v.at[:, pl.ds(0, _ROWS_W)], i_sem
    )
    c_pe = pltpu.async_copy(pe_hbm.at[pl.ds(l0, _ROWS_W)], pe_v, i_sem)
    c_tab.wait()
    c_x.wait()
    c_pe.wait()

    def batch_body(b, carry2):
        par = b % 2

        @pl.when(b >= 2)
        def _wait_older_store():
            pltpu.make_async_copy(
                o_v.at[par], out_hbm.at[b, pl.ds(l0, _ROWS_W)], o_sem
            ).wait()

        w16 = x_v[b, pl.ds(0, _LANES)].astype(jnp.float32)
        for dc in range(0, _ND, _DC):
            diffs = [
                tb_v[1, pl.ds((dc + j) * _LANES, _LANES)]
                - tb_v[0, pl.ds((dc + j) * _LANES, _LANES)]
                for j in range(_DC)
            ]

            def row_body(r, carry3, dc=dc, diffs=diffs, w16=w16):
                wv = lax.gather(
                    w16,
                    jnp.full((_LANES, 1), r, jnp.int32),
                    dimension_numbers=lax.GatherDimensionNumbers(
                        offset_dims=(),
                        collapsed_slice_dims=(0,),
                        start_index_map=(0,),
                    ),
                    slice_sizes=(1,),
                    mode=lax.GatherScatterMode.PROMISE_IN_BOUNDS,
                )
                pes = [
                    pe_v[r, pl.ds((dc + j) * _LANES, _LANES)] for j in range(_DC)
                ]
                outs = [pes[j] + wv * diffs[j] for j in range(_DC)]
                for j in range(_DC):
                    o_v[par, r, pl.ds((dc + j) * _LANES, _LANES)] = outs[j]
                return carry3

            lax.fori_loop(0, _ROWS_W, row_body, 0)
        pltpu.async_copy(o_v.at[par], out_hbm.at[b, pl.ds(l0, _ROWS_W)], o_sem)
        return carry2

    lax.fori_loop(0, _BATCH, batch_body, 0)
    for _ in range(2):
        pltpu.make_async_copy(
            o_v.at[0], out_hbm.at[0, pl.ds(l0, _ROWS_W)], o_sem
        ).wait()


_LB = 512
_TC_OFF = _S_SC // _LB


def _tc_body(x_ref, tab_ref, pe_ref, alias_ref, out_ref):
    row0 = tab_ref[0, :]
    diff = tab_ref[1, :] - row0
    base = pe_ref[:, :] + row0[None, :]
    w = x_ref[:, :].astype(jnp.float32)[:, :, None]
    out_ref[:, :, :] = base[None, :, :] + w * diff[None, None, :]


def kernel(x, embed_table):
    pe0_sc = jnp.asarray(_PE[:_S_SC]) + embed_table[0][None, :]
    sc_out = _sc_embed(x, embed_table, pe0_sc)
    n_tc = (_SEQ - _S_SC) // _LB
    return pl.pallas_call(
        _tc_body,
        grid=(n_tc,),
        in_specs=[
            pl.BlockSpec((_BATCH, _LB), lambda i: (0, i + _TC_OFF)),
            pl.BlockSpec((2, _D_MODEL), lambda i: (0, 0)),
            pl.BlockSpec((_LB, _D_MODEL), lambda i: (i + _TC_OFF, 0)),
            pl.BlockSpec(memory_space=pl.ANY),
        ],
        out_specs=pl.BlockSpec(
            (_BATCH, _LB, _D_MODEL), lambda i: (0, i + _TC_OFF, 0)
        ),
        out_shape=jax.ShapeDtypeStruct((_BATCH, _SEQ, _D_MODEL), jnp.float32),
        input_output_aliases={3: 0},
    )(x, embed_table, _PE, sc_out)


# speedup vs baseline: 1.0457x; 1.0457x over previous
import functools
import math

import jax
import jax.numpy as jnp
import numpy as np
from jax import lax
from jax.experimental import pallas as pl
from jax.experimental.pallas import tpu as pltpu
from jax.experimental.pallas import tpu_sc as plsc

_D_MODEL = 1024
_MAX_LEN = 2048
_BATCH = 4
_SEQ = 2048


def _pe_table(max_len, d_model):
    pe = np.zeros((max_len, d_model), dtype=np.float32)
    pos = np.arange(max_len, dtype=np.float32)[:, None]
    div = np.exp(
        np.arange(0, d_model, 2, dtype=np.float32) * (-math.log(10000.0) / d_model)
    )
    pe[:, 0::2] = np.sin(pos * div)
    pe[:, 1::2] = np.cos(pos * div[: d_model // 2])
    return pe


_PE = _pe_table(_MAX_LEN, _D_MODEL)

_NC = 2
_NS = 16
_NW = _NC * _NS
_LANES = 16
_S_SC = 256
_ROWS_W = _S_SC // _NW
_ND = _D_MODEL // _LANES
_DC = 16

_mesh = plsc.VectorSubcoreMesh(core_axis_name="c", subcore_axis_name="s")


@functools.partial(
    pl.kernel,
    mesh=_mesh,
    out_type=jax.ShapeDtypeStruct((_BATCH, _SEQ, _D_MODEL), jnp.float32),
    scratch_types=[
        pltpu.VMEM((2, _D_MODEL), jnp.float32),
        pltpu.VMEM((_BATCH, _ROWS_W + _LANES), jnp.int32),
        pltpu.VMEM((_ROWS_W, _D_MODEL), jnp.float32),
        pltpu.VMEM((2, _ROWS_W, _D_MODEL), jnp.float32),
        pltpu.SemaphoreType.DMA,
        pltpu.SemaphoreType.DMA,
    ],
)
def _sc_embed(x_hbm, tab_hbm, pe_hbm, out_hbm, tb_v, x_v, pe_v, o_v, o_sem, i_sem):
    wid = lax.axis_index("s") * _NC + lax.axis_index("c")
    l0 = wid * _ROWS_W
    c_tab = pltpu.async_copy(tab_hbm, tb_v, i_sem)
    c_x = pltpu.async_copy(
        x_hbm.at[:, pl.ds(l0, _ROWS_W)], x_v.at[:, pl.ds(0, _ROWS_W)], i_sem
    )
    c_pe = pltpu.async_copy(pe_hbm.at[pl.ds(l0, _ROWS_W)], pe_v, i_sem)
    c_tab.wait()
    c_x.wait()
    c_pe.wait()

    def batch_body(b, carry2):
        par = b % 2

        @pl.when(b >= 2)
        def _wait_older_store():
            pltpu.make_async_copy(
                o_v.at[par], out_hbm.at[b, pl.ds(l0, _ROWS_W)], o_sem
            ).wait()

        w16 = x_v[b, pl.ds(0, _LANES)].astype(jnp.float32)
        for dc in range(0, _ND, _DC):
            diffs = [
                tb_v[1, pl.ds((dc + j) * _LANES, _LANES)]
                - tb_v[0, pl.ds((dc + j) * _LANES, _LANES)]
                for j in range(_DC)
            ]

            def row_body(r, carry3, dc=dc, diffs=diffs, w16=w16):
                wv = lax.gather(
                    w16,
                    jnp.full((_LANES, 1), r, jnp.int32),
                    dimension_numbers=lax.GatherDimensionNumbers(
                        offset_dims=(),
                        collapsed_slice_dims=(0,),
                        start_index_map=(0,),
                    ),
                    slice_sizes=(1,),
                    mode=lax.GatherScatterMode.PROMISE_IN_BOUNDS,
                )
                pes = [
                    pe_v[r, pl.ds((dc + j) * _LANES, _LANES)] for j in range(_DC)
                ]
                outs = [pes[j] + wv * diffs[j] for j in range(_DC)]
                for j in range(_DC):
                    o_v[par, r, pl.ds((dc + j) * _LANES, _LANES)] = outs[j]
                return carry3

            lax.fori_loop(0, _ROWS_W, row_body, 0)
        pltpu.async_copy(o_v.at[par], out_hbm.at[b, pl.ds(l0, _ROWS_W)], o_sem)
        return carry2

    lax.fori_loop(0, _BATCH, batch_body, 0)
    for _ in range(2):
        pltpu.make_async_copy(
            o_v.at[0], out_hbm.at[0, pl.ds(l0, _ROWS_W)], o_sem
        ).wait()


_LB = 256
_TC_OFF = _S_SC // _LB


def _tc_body(x_ref, tab_ref, pe_ref, alias_ref, out_ref):
    row0 = tab_ref[0, :]
    diff = tab_ref[1, :] - row0
    base = pe_ref[:, :] + row0[None, :]
    w = x_ref[:, :].astype(jnp.float32)[:, :, None]
    out_ref[:, :, :] = base[None, :, :] + w * diff[None, None, :]


def kernel(x, embed_table):
    pe0_sc = jnp.asarray(_PE[:_S_SC]) + embed_table[0][None, :]
    sc_out = _sc_embed(x, embed_table, pe0_sc)
    n_tc = (_SEQ - _S_SC) // _LB
    return pl.pallas_call(
        _tc_body,
        grid=(n_tc,),
        in_specs=[
            pl.BlockSpec((_BATCH, _LB), lambda i: (0, i + _TC_OFF)),
            pl.BlockSpec((2, _D_MODEL), lambda i: (0, 0)),
            pl.BlockSpec((_LB, _D_MODEL), lambda i: (i + _TC_OFF, 0)),
            pl.BlockSpec(memory_space=pl.ANY),
        ],
        out_specs=pl.BlockSpec(
            (_BATCH, _LB, _D_MODEL), lambda i: (0, i + _TC_OFF, 0)
        ),
        out_shape=jax.ShapeDtypeStruct((_BATCH, _SEQ, _D_MODEL), jnp.float32),
        input_output_aliases={3: 0},
    )(x, embed_table, _PE, sc_out)
